# baseline (speedup 1.0000x reference)
import functools

import jax
import jax.numpy as jnp
from jax import lax
from jax.experimental import pallas as pl
from jax.experimental.pallas import tpu as pltpu
from jax.experimental.pallas import tpu_sc as plsc

_DIM = 64
_LANES = 16
_NC = 2
_NS = 16
_NW = _NC * _NS
_BLK = 128
_NBUF = 2


@functools.lru_cache(maxsize=None)
def _build(n_k, n_b):
    blocks_per_k = n_b // _BLK
    n_blocks = n_k * blocks_per_k
    blocks_per_w = n_blocks // _NW
    mesh = plsc.VectorSubcoreMesh(core_axis_name="c", subcore_axis_name="s",
                                  num_cores=_NC, num_subcores=_NS)

    @functools.partial(
        pl.kernel,
        out_type=jax.ShapeDtypeStruct((n_k, _DIM, n_b), jnp.float32),
        mesh=mesh,
        scratch_types=[
            *[pltpu.VMEM((_BLK,), jnp.int32) for _ in range(3 * _NBUF)],
            *[pltpu.VMEM((_BLK, 2 * _DIM), jnp.float32) for _ in range(_NBUF)],
            *[pltpu.VMEM((_DIM, _BLK), jnp.float32) for _ in range(2 * _NBUF)],
            *[pltpu.SemaphoreType.DMA for _ in range(2 * _NBUF)],
        ],
        compiler_params=pltpu.CompilerParams(needs_layout_passes=False),
    )
    def body(slots_hbm, idx_hbm, table_hbm, out_hbm,
             idx0, idx1, pair0, pair1, par0, par1,
             rows0, rows1, slots0, slots1, outv0, outv1,
             gs0, gs1, os0, os1):
        idx_v = [idx0, idx1]
        pair_v = [pair0, pair1]
        par_v = [par0, par1]
        rows_v = [rows0, rows1]
        slots_v = [slots0, slots1]
        out_v = [outv0, outv1]
        gsem = [gs0, gs1]
        osem = [os0, os1]
        wid = lax.axis_index("s") * _NC + lax.axis_index("c")
        base = wid * blocks_per_w
        iota16 = lax.iota(jnp.int32, _LANES)

        def coords(t):
            beta = base + t
            return beta // blocks_per_k, (beta % blocks_per_k) * _BLK

        def gather_copy(t, b):
            return pltpu.make_async_copy(
                table_hbm.at[pair_v[b]], rows_v[b], gsem[b])

        def slots_copy(t, b):
            k, b0 = coords(t)
            return pltpu.make_async_copy(
                slots_hbm.at[k, :, pl.ds(b0, _BLK)], slots_v[b], gsem[b])

        def store_copy(t, b):
            k, b0 = coords(t)
            return pltpu.make_async_copy(
                out_v[b], out_hbm.at[k, :, pl.ds(b0, _BLK)], osem[b])

        def prep(t, b):
            k, b0 = coords(t)
            pltpu.sync_copy(idx_hbm.at[k, pl.ds(b0, _BLK)], idx_v[b])
            for v in range(_BLK // _LANES):
                sl = pl.ds(v * _LANES, _LANES)
                s = idx_v[b][sl]
                pair_v[b][sl] = lax.shift_right_logical(s, 1)
                par_v[b][sl] = lax.shift_left(s & 1, 6)
            gather_copy(t, b).start()
            slots_copy(t, b).start()

        for b in range(_NBUF):
            prep(b, b)

        def outer(g, carry):
            for b in range(_NBUF):
                t = g * _NBUF + b
                gather_copy(t, b).wait()
                slots_copy(t, b).wait()

                @pl.when(t >= _NBUF)
                def _():
                    store_copy(t - _NBUF, b).wait()

                for bb in range(_BLK // _LANES):
                    sl = pl.ds(bb * _LANES, _LANES)
                    par_bb = par_v[b][sl]
                    rows_bb = iota16 + (bb * _LANES)

                    @pl.loop(0, _DIM)
                    def _(d, par_bb=par_bb, rows_bb=rows_bb, sl=sl, b=b):
                        colv = plsc.load_gather(
                            rows_v[b], [rows_bb, par_bb + d])
                        out_v[b][d, sl] = colv * slots_v[b][d, sl]

                store_copy(t, b).start()

                @pl.when(t + _NBUF < blocks_per_w)
                def _():
                    prep(t + _NBUF, b)
            return carry

        lax.fori_loop(0, blocks_per_w // _NBUF, outer, 0)
        for b in range(_NBUF):
            store_copy(blocks_per_w - _NBUF + b, b).wait()

    return body


def kernel(slots, indices, scale_values):
    b, k, d = slots.shape
    n_rows, _ = scale_values.shape
    slots_t = jnp.transpose(slots, (1, 2, 0))
    idx_t = jnp.transpose(indices.astype(jnp.int32))
    table2 = scale_values.reshape(n_rows // 2, 2 * d)
    out_t = _build(k, b)(slots_t, idx_t, table2)
    return jnp.transpose(out_t, (2, 0, 1))
